# baseline (speedup 1.0000x reference)
import functools

import jax
import jax.numpy as jnp
from jax import lax
from jax.experimental import pallas as pl
from jax.experimental.pallas import tpu as pltpu
from jax.experimental.pallas import tpu_sc as plsc

_LANES = 16


def _build_sc_kernel(C, H, Wl, Wr, D, interpret=False):
    mesh = plsc.VectorSubcoreMesh(
        core_axis_name="c", subcore_axis_name="s", num_cores=2, num_subcores=16
    )
    n_workers = mesh.num_cores * mesh.num_subcores
    assert C == n_workers

    groups = Wl // _LANES

    @functools.partial(
        pl.kernel,
        out_type=jax.ShapeDtypeStruct((C, D, H * Wl), jnp.float32),
        mesh=mesh,
        scratch_types=[
            pltpu.VMEM((H * Wl,), jnp.float32),
            pltpu.VMEM((H * Wr,), jnp.float32),
            pltpu.VMEM((H * Wl,), jnp.float32),
        ],
        compiler_params=pltpu.CompilerParams(needs_layout_passes=False),
        interpret=interpret,
    )
    def k(left_hbm, right_hbm, out_hbm, left_v, right_v, out_v):
        wid = lax.axis_index("s") * mesh.num_cores + lax.axis_index("c")
        pltpu.sync_copy(left_hbm.at[wid], left_v)
        pltpu.sync_copy(right_hbm.at[wid], right_v)
        lane4 = lax.iota(jnp.int32, _LANES) * 4

        def d_body(d, carry):
            def h_body(h, carry2):
                rbase = h * Wr
                obase = h * Wl
                for j in range(groups):
                    lft = left_v[pl.ds(obase + j * _LANES, _LANES)]
                    idx = jnp.maximum(lane4 + (j * _LANES * 4 + 1) - d, 0) + rbase
                    rgt = plsc.load_gather(right_v, [idx])
                    out_v[pl.ds(obase + j * _LANES, _LANES)] = lft - rgt
                return carry2

            lax.fori_loop(0, H, h_body, 0, unroll=False)
            pltpu.sync_copy(out_v, out_hbm.at[wid, d])
            return carry

        lax.fori_loop(0, D, d_body, 0, unroll=False)

    return k


def kernel(left_feature, right_feature, max_disp):
    B, C, H, Wl = left_feature.shape
    Wr = right_feature.shape[3]
    D = 48
    left2 = left_feature.reshape(C, H * Wl)
    right2 = right_feature.reshape(C, H * Wr)
    k = _build_sc_kernel(C, H, Wl, Wr, D)
    out = k(left2, right2)
    return out.reshape(B, C, D, H, Wl)


# speedup vs baseline: 1.0002x; 1.0002x over previous
import functools

import jax
import jax.numpy as jnp
from jax import lax
from jax.experimental import pallas as pl
from jax.experimental.pallas import tpu as pltpu
from jax.experimental.pallas import tpu_sc as plsc

_LANES = 16


def _build_sc_kernel(C, H, Wl, Wr, D, interpret=False):
    mesh = plsc.VectorSubcoreMesh(
        core_axis_name="c", subcore_axis_name="s", num_cores=2, num_subcores=16
    )
    n_workers = mesh.num_cores * mesh.num_subcores
    assert C == n_workers

    groups = Wl // _LANES

    @functools.partial(
        pl.kernel,
        out_type=jax.ShapeDtypeStruct((C, D, H * Wl), jnp.float32),
        mesh=mesh,
        scratch_types=[
            pltpu.VMEM((H * Wl,), jnp.float32),
            pltpu.VMEM((H * Wr,), jnp.float32),
            pltpu.VMEM((2, H * Wl), jnp.float32),
            pltpu.SemaphoreType.DMA,
            pltpu.SemaphoreType.DMA,
        ],
        compiler_params=pltpu.CompilerParams(needs_layout_passes=False),
        interpret=interpret,
    )
    def k(left_hbm, right_hbm, out_hbm, left_v, right_v, out_v, sem0, sem1):
        wid = lax.axis_index("s") * mesh.num_cores + lax.axis_index("c")
        pltpu.sync_copy(left_hbm.at[wid], left_v)
        pltpu.sync_copy(right_hbm.at[wid], right_v)
        lane4 = lax.iota(jnp.int32, _LANES) * 4
        sems = (sem0, sem1)

        def compute_slab(d, buf):
            bvec = [
                jnp.maximum(lane4 + (j * _LANES * 4 + 1) - d, 0)
                for j in range(groups)
            ]

            def h_body(h, carry2):
                rbase = h * Wr
                obase = h * Wl
                for j in range(groups):
                    lft = left_v[pl.ds(obase + j * _LANES, _LANES)]
                    rgt = plsc.load_gather(right_v, [bvec[j] + rbase])
                    out_v[buf, pl.ds(obase + j * _LANES, _LANES)] = lft - rgt
                return carry2

            lax.fori_loop(0, H, h_body, 0, unroll=False)

        def d2_body(dd, carry):
            for buf in range(2):
                d = dd * 2 + buf
                @pl.when(dd > 0)
                def _():
                    pltpu.make_async_copy(
                        out_v.at[buf], out_hbm.at[wid, d], sems[buf]
                    ).wait()

                compute_slab(d, buf)
                pltpu.async_copy(out_v.at[buf], out_hbm.at[wid, d], sems[buf])
            return carry

        lax.fori_loop(0, D // 2, d2_body, 0, unroll=False)
        for buf in range(2):
            pltpu.make_async_copy(
                out_v.at[buf], out_hbm.at[wid, D - 2 + buf], sems[buf]
            ).wait()

    return k


def kernel(left_feature, right_feature, max_disp):
    B, C, H, Wl = left_feature.shape
    Wr = right_feature.shape[3]
    D = 48
    left2 = left_feature.reshape(C, H * Wl)
    right2 = right_feature.reshape(C, H * Wr)
    k = _build_sc_kernel(C, H, Wl, Wr, D)
    out = k(left2, right2)
    return out.reshape(B, C, D, H, Wl)


# speedup vs baseline: 1.0019x; 1.0018x over previous
import functools

import jax
import jax.numpy as jnp
from jax import lax
from jax.experimental import pallas as pl
from jax.experimental.pallas import tpu as pltpu
from jax.experimental.pallas import tpu_sc as plsc

_LANES = 16


def _build_sc_kernel(C, H, Wl, Wr, D, interpret=False):
    mesh = plsc.VectorSubcoreMesh(
        core_axis_name="c", subcore_axis_name="s", num_cores=2, num_subcores=16
    )
    n_workers = mesh.num_cores * mesh.num_subcores
    assert C == n_workers

    groups = Wl // _LANES

    @functools.partial(
        pl.kernel,
        out_type=jax.ShapeDtypeStruct((C, D, H * Wl), jnp.float32),
        mesh=mesh,
        scratch_types=[
            pltpu.VMEM((H * Wl,), jnp.float32),
            pltpu.VMEM((H * Wr,), jnp.float32),
            pltpu.VMEM((2, H * Wl), jnp.float32),
            pltpu.SemaphoreType.DMA,
            pltpu.SemaphoreType.DMA,
        ],
        compiler_params=pltpu.CompilerParams(needs_layout_passes=False),
        interpret=interpret,
    )
    def k(left_hbm, right_hbm, out_hbm, left_v, right_v, out_v, sem0, sem1):
        wid = lax.axis_index("s") * mesh.num_cores + lax.axis_index("c")
        pltpu.sync_copy(left_hbm.at[wid], left_v)
        pltpu.sync_copy(right_hbm.at[wid], right_v)
        lane4 = lax.iota(jnp.int32, _LANES) * 4
        sems = (sem0, sem1)

        def compute_slab(d, buf):
            bvec = [
                jnp.maximum(lane4 + (j * _LANES * 4 + 1) - d, 0)
                for j in range(groups)
            ]

            def h_body(h, carry2):
                rbase = h * Wr
                obase = h * Wl
                for j in range(groups):
                    lft = left_v[pl.ds(obase + j * _LANES, _LANES)]
                    rgt = plsc.load_gather(right_v, [bvec[j] + rbase])
                    out_v[buf, pl.ds(obase + j * _LANES, _LANES)] = lft - rgt
                return carry2

            lax.fori_loop(0, H, h_body, 0, unroll=4)

        def d2_body(dd, carry):
            for buf in range(2):
                d = dd * 2 + buf
                @pl.when(dd > 0)
                def _():
                    pltpu.make_async_copy(
                        out_v.at[buf], out_hbm.at[wid, d], sems[buf]
                    ).wait()

                compute_slab(d, buf)
                pltpu.async_copy(out_v.at[buf], out_hbm.at[wid, d], sems[buf])
            return carry

        lax.fori_loop(0, D // 2, d2_body, 0, unroll=False)
        for buf in range(2):
            pltpu.make_async_copy(
                out_v.at[buf], out_hbm.at[wid, D - 2 + buf], sems[buf]
            ).wait()

    return k


def kernel(left_feature, right_feature, max_disp):
    B, C, H, Wl = left_feature.shape
    Wr = right_feature.shape[3]
    D = 48
    left2 = left_feature.reshape(C, H * Wl)
    right2 = right_feature.reshape(C, H * Wr)
    k = _build_sc_kernel(C, H, Wl, Wr, D)
    out = k(left2, right2)
    return out.reshape(B, C, D, H, Wl)


# speedup vs baseline: 1.3617x; 1.3590x over previous
import functools

import jax
import jax.numpy as jnp
from jax import lax
from jax.experimental import pallas as pl
from jax.experimental.pallas import tpu as pltpu
from jax.experimental.pallas import tpu_sc as plsc

_LANES = 16


def _build_sc_kernel(C, H, Wl, Wr, D, interpret=False):
    mesh = plsc.VectorSubcoreMesh(
        core_axis_name="c", subcore_axis_name="s", num_cores=2, num_subcores=16
    )
    n_workers = mesh.num_cores * mesh.num_subcores
    assert C == n_workers

    groups = Wl // _LANES
    r_rows = H * Wr // Wl

    @functools.partial(
        pl.kernel,
        out_type=jax.ShapeDtypeStruct((C, D * H, Wl), jnp.float32),
        mesh=mesh,
        scratch_types=[
            pltpu.VMEM((H, Wl), jnp.float32),
            pltpu.VMEM((r_rows, Wl), jnp.float32),
            pltpu.VMEM((2, H, Wl), jnp.float32),
            pltpu.SemaphoreType.DMA,
            pltpu.SemaphoreType.DMA,
        ],
        compiler_params=pltpu.CompilerParams(needs_layout_passes=False),
        interpret=interpret,
    )
    def k(left_hbm, right_hbm, out_hbm, left_v, right_v, out_v, sem0, sem1):
        wid = lax.axis_index("s") * mesh.num_cores + lax.axis_index("c")
        pltpu.sync_copy(left_hbm.at[wid], left_v)
        pltpu.sync_copy(right_hbm.at[wid], right_v)
        lane4 = lax.iota(jnp.int32, _LANES) * 4
        sems = (sem0, sem1)

        def compute_slab(d, buf):
            bvec = [
                jnp.maximum(lane4 + (j * _LANES * 4 + 1) - d, 0)
                for j in range(groups)
            ]

            def h_body(h, carry2):
                rbase = h * Wr
                for j in range(groups):
                    lft = left_v[h, pl.ds(j * _LANES, _LANES)]
                    flat = bvec[j] + rbase
                    rgt = plsc.load_gather(
                        right_v,
                        [lax.shift_right_logical(flat, 7), flat & (Wl - 1)],
                    )
                    out_v[buf, h, pl.ds(j * _LANES, _LANES)] = lft - rgt
                return carry2

            lax.fori_loop(0, H, h_body, 0, unroll=2)

        def d2_body(dd, carry):
            for buf in range(2):
                d = dd * 2 + buf
                @pl.when(dd > 0)
                def _():
                    pltpu.make_async_copy(
                        out_v.at[buf], out_hbm.at[wid, pl.ds(d * H, H)], sems[buf]
                    ).wait()

                compute_slab(d, buf)
                pltpu.async_copy(
                    out_v.at[buf], out_hbm.at[wid, pl.ds(d * H, H)], sems[buf]
                )
            return carry

        lax.fori_loop(0, D // 2, d2_body, 0, unroll=False)
        for buf in range(2):
            pltpu.make_async_copy(
                out_v.at[buf],
                out_hbm.at[wid, pl.ds((D - 2 + buf) * H, H)],
                sems[buf],
            ).wait()

    return k


def kernel(left_feature, right_feature, max_disp):
    B, C, H, Wl = left_feature.shape
    Wr = right_feature.shape[3]
    D = 48
    left3 = left_feature.reshape(C, H, Wl)
    right3 = right_feature.reshape(C, H * Wr // Wl, Wl)
    k = _build_sc_kernel(C, H, Wl, Wr, D)
    out = k(left3, right3)
    return out.reshape(B, C, D, H, Wl)


# speedup vs baseline: 3.2701x; 2.4015x over previous
import functools

import jax
import jax.numpy as jnp
from jax import lax
from jax.experimental import pallas as pl
from jax.experimental.pallas import tpu as pltpu
from jax.experimental.pallas import tpu_sc as plsc

_LANES = 16


def _build_sc_kernel(C, H, Wl, Wr, D, interpret=False):
    mesh = plsc.VectorSubcoreMesh(
        core_axis_name="c", subcore_axis_name="s", num_cores=2, num_subcores=16
    )
    n_workers = mesh.num_cores * mesh.num_subcores
    assert C == n_workers

    groups = Wl // _LANES
    r_rows = H * Wr // Wl

    @functools.partial(
        pl.kernel,
        out_type=jax.ShapeDtypeStruct((C, D * H, Wl), jnp.float32),
        mesh=mesh,
        scratch_types=[
            pltpu.VMEM((H, Wl), jnp.float32),
            pltpu.VMEM((r_rows, Wl), jnp.float32),
            pltpu.VMEM((2, H, Wl), jnp.float32),
            pltpu.SemaphoreType.DMA,
            pltpu.SemaphoreType.DMA,
        ],
        compiler_params=pltpu.CompilerParams(needs_layout_passes=False),
        interpret=interpret,
    )
    def k(left_hbm, right_hbm, out_hbm, left_v, right_v, out_v, sem0, sem1):
        wid = lax.axis_index("s") * mesh.num_cores + lax.axis_index("c")
        pltpu.sync_copy(left_hbm.at[wid], left_v)
        pltpu.sync_copy(right_hbm.at[wid], right_v)
        lane4 = lax.iota(jnp.int32, _LANES) * 4
        sems = (sem0, sem1)

        def compute_slab(d, buf):
            bvec = [
                jnp.maximum(lane4 + (j * _LANES * 4 + 1) - d, 0)
                for j in range(groups)
            ]

            def h_body(h, carry2):
                rbase = h * Wr
                lfts = [left_v[h, pl.ds(j * _LANES, _LANES)] for j in range(groups)]
                rgts = []
                for j in range(groups):
                    flat = bvec[j] + rbase
                    rgts.append(
                        plsc.load_gather(
                            right_v,
                            [lax.shift_right_logical(flat, 7), flat & (Wl - 1)],
                        )
                    )
                for j in range(groups):
                    out_v[buf, h, pl.ds(j * _LANES, _LANES)] = lfts[j] - rgts[j]
                return carry2

            lax.fori_loop(0, H, h_body, 0, unroll=2)

        def d2_body(dd, carry):
            for buf in range(2):
                d = dd * 2 + buf
                @pl.when(dd > 0)
                def _():
                    pltpu.make_async_copy(
                        out_v.at[buf], out_hbm.at[wid, pl.ds(d * H, H)], sems[buf]
                    ).wait()

                compute_slab(d, buf)
                pltpu.async_copy(
                    out_v.at[buf], out_hbm.at[wid, pl.ds(d * H, H)], sems[buf]
                )
            return carry

        lax.fori_loop(0, D // 2, d2_body, 0, unroll=False)
        for buf in range(2):
            pltpu.make_async_copy(
                out_v.at[buf],
                out_hbm.at[wid, pl.ds((D - 2 + buf) * H, H)],
                sems[buf],
            ).wait()

    return k


def kernel(left_feature, right_feature, max_disp):
    B, C, H, Wl = left_feature.shape
    Wr = right_feature.shape[3]
    D = 48
    left3 = left_feature.reshape(C, H, Wl)
    right3 = right_feature.reshape(C, H * Wr // Wl, Wl)
    k = _build_sc_kernel(C, H, Wl, Wr, D)
    out = k(left3, right3)
    return out.reshape(B, C, D, H, Wl)


# speedup vs baseline: 3.2867x; 1.0051x over previous
import functools

import jax
import jax.numpy as jnp
from jax import lax
from jax.experimental import pallas as pl
from jax.experimental.pallas import tpu as pltpu
from jax.experimental.pallas import tpu_sc as plsc

_LANES = 16
_DB = 4


def _build_sc_kernel(C, H, Wl, Wr, D, interpret=False):
    mesh = plsc.VectorSubcoreMesh(
        core_axis_name="c", subcore_axis_name="s", num_cores=2, num_subcores=16
    )
    n_workers = mesh.num_cores * mesh.num_subcores
    assert C == n_workers and D % (2 * _DB) == 0

    groups = Wl // _LANES
    r_rows = H * Wr // Wl
    n_blocks = D // _DB

    @functools.partial(
        pl.kernel,
        out_type=jax.ShapeDtypeStruct((C, D * H, Wl), jnp.float32),
        mesh=mesh,
        scratch_types=[
            pltpu.VMEM((H, Wl), jnp.float32),
            pltpu.VMEM((r_rows, Wl), jnp.float32),
            pltpu.VMEM((2, _DB * H, Wl), jnp.float32),
            pltpu.SemaphoreType.DMA,
            pltpu.SemaphoreType.DMA,
        ],
        compiler_params=pltpu.CompilerParams(needs_layout_passes=False),
        interpret=interpret,
    )
    def k(left_hbm, right_hbm, out_hbm, left_v, right_v, out_v, sem0, sem1):
        wid = lax.axis_index("s") * mesh.num_cores + lax.axis_index("c")
        pltpu.sync_copy(left_hbm.at[wid], left_v)
        pltpu.sync_copy(right_hbm.at[wid], right_v)
        lane = lax.iota(jnp.int32, _LANES)
        base = [lane * 4 + j * _LANES * 4 for j in range(groups)]
        sems = (sem0, sem1)

        def compute_block(m, buf):
            d0 = m * _DB

            def h_body(h, carry2):
                rbase = h * Wr
                lfts = [left_v[h, pl.ds(j * _LANES, _LANES)] for j in range(groups)]
                for r in range(_DB):
                    off = rbase + 1 - r - d0
                    flat0 = jnp.maximum(base[0] + (1 - r) - d0, 0) + rbase
                    flats = [flat0] + [base[j] + off for j in range(1, groups)]
                    rgts = [
                        plsc.load_gather(
                            right_v,
                            [lax.shift_right_logical(f, 7), f & (Wl - 1)],
                        )
                        for f in flats
                    ]
                    orow = r * H + h
                    for j in range(groups):
                        out_v[buf, orow, pl.ds(j * _LANES, _LANES)] = (
                            lfts[j] - rgts[j]
                        )
                return carry2

            lax.fori_loop(0, H, h_body, 0, unroll=1)

        def m2_body(mm, carry):
            for buf in range(2):
                m = mm * 2 + buf
                @pl.when(mm > 0)
                def _():
                    pltpu.make_async_copy(
                        out_v.at[buf],
                        out_hbm.at[wid, pl.ds(m * _DB * H, _DB * H)],
                        sems[buf],
                    ).wait()

                compute_block(m, buf)
                pltpu.async_copy(
                    out_v.at[buf],
                    out_hbm.at[wid, pl.ds(m * _DB * H, _DB * H)],
                    sems[buf],
                )
            return carry

        lax.fori_loop(0, n_blocks // 2, m2_body, 0, unroll=False)
        for buf in range(2):
            pltpu.make_async_copy(
                out_v.at[buf],
                out_hbm.at[wid, pl.ds((n_blocks - 2 + buf) * _DB * H, _DB * H)],
                sems[buf],
            ).wait()

    return k


def kernel(left_feature, right_feature, max_disp):
    B, C, H, Wl = left_feature.shape
    Wr = right_feature.shape[3]
    D = 48
    left3 = left_feature.reshape(C, H, Wl)
    right3 = right_feature.reshape(C, H * Wr // Wl, Wl)
    k = _build_sc_kernel(C, H, Wl, Wr, D)
    out = k(left3, right3)
    return out.reshape(B, C, D, H, Wl)
